# Initial kernel scaffold; baseline (speedup 1.0000x reference)
import jax
import jax.numpy as jnp
from jax.experimental import pallas as pl


def kernel(atomic_number, r, edge_index, t_src, t_dst, params):
    raise NotImplementedError("write your pallas kernel here")


# speedup vs baseline: 98.4246x; 98.4246x over previous
import functools

import jax
import jax.numpy as jnp
from jax.experimental import pallas as pl
from jax.experimental.pallas import tpu as pltpu

DEG = 16
CHEB = 64
DMSG = 64
DMODEL = 128
BLOCK_N = 128


def _triplet_kernel(attn_ref, xij_ref, rn_ref, out_ref):
    B = out_ref.shape[1]
    r0 = rn_ref[0]
    r1 = rn_ref[1]
    r2 = rn_ref[2]
    jidx = jax.lax.broadcasted_iota(jnp.int32, (DEG, B), 0)

    acol = jnp.zeros((DEG, B), jnp.float32)
    for i in range(DEG):
        ci = r0[i : i + 1] * r0 + r1[i : i + 1] * r1 + r2[i : i + 1] * r2

        def chan(k, t, acc):
            xk = xij_ref[k]
            arg = t + xk[i : i + 1] + xk
            e = arg * (1.0 / (1.0 + jnp.exp(-arg)))
            return acc + attn_ref[k] * e

        acc = chan(0, jnp.ones((DEG, B), jnp.float32), jnp.zeros((DEG, B), jnp.float32))
        acc = chan(1, ci, acc)

        def body(k, carry):
            tprev, tcur, acc = carry
            tnew = 2.0 * ci * tcur - tprev
            return (tcur, tnew, chan(k, tnew, acc))

        _, _, logit = jax.lax.fori_loop(2, CHEB, body, (jnp.ones((DEG, B), jnp.float32), ci, acc))

        li = jnp.where(jidx == i, -1e30, logit)
        m = jnp.max(li, axis=0, keepdims=True)
        w = jnp.exp(li - m)
        denom = jnp.sum(w, axis=0, keepdims=True)
        acol = acol + w / denom

    x_all = xij_ref[...]
    out_ref[...] = jnp.sum(x_all * acol[None], axis=1)


def _triplet_call(attn, xij_t, rn_t):
    np_ = xij_t.shape[2]
    grid = np_ // BLOCK_N
    return pl.pallas_call(
        _triplet_kernel,
        grid=(grid,),
        in_specs=[
            pl.BlockSpec(memory_space=pltpu.SMEM),
            pl.BlockSpec((CHEB, DEG, BLOCK_N), lambda nb: (0, 0, nb)),
            pl.BlockSpec((3, DEG, BLOCK_N), lambda nb: (0, 0, nb)),
        ],
        out_specs=pl.BlockSpec((DMSG, BLOCK_N), lambda nb: (0, nb)),
        out_shape=jax.ShapeDtypeStruct((DMSG, np_), jnp.float32),
    )(attn, xij_t, rn_t)


def kernel(atomic_number, r, edge_index, t_src, t_dst, params):
    src = edge_index[0].astype(jnp.int32)
    n = atomic_number.shape[0]
    e = r.shape[0]
    npad = ((n + BLOCK_N - 1) // BLOCK_N) * BLOCK_N

    x = jnp.take(params["emb"], atomic_number, axis=0)

    bl = jnp.sqrt(jnp.sum(r * r, axis=1))
    centers = jnp.linspace(0.0, 8.0, DMODEL)
    gamma = 1.0 / (centers[1] - centers[0])
    y = jnp.exp(-gamma * (bl[:, None] - centers[None, :]) ** 2)
    rnorm = -r / bl[:, None]

    rn_t = jnp.transpose(rnorm.reshape(n, DEG, 3), (2, 1, 0))
    rn_t = jnp.pad(rn_t, ((0, 0), (0, 0), (0, npad - n)))

    for p in params["layers"]:
        xs = x @ p["Wsrc"] + p["bsrc"]
        xd = x @ p["Wdst"] + p["bdst"]
        ye = y @ p["Wedge"] + p["bedge"]
        xij = xs[src] + jnp.repeat(xd, DEG, axis=0) + ye
        xij_t = jnp.transpose(xij.reshape(n, DEG, DMSG), (2, 1, 0))
        xij_t = jnp.pad(xij_t, ((0, 0), (0, 0), (0, npad - n)))

        ft_t = _triplet_call(p["attn"][0], xij_t, rn_t)
        ft = ft_t[:, :n].T

        h = jax.nn.silu(ft @ p["W1"] + p["b1"])
        x = h @ p["W2"] + p["b2"]

    atom_e = x @ params["Wf"] + params["bf"]
    return jnp.squeeze(jnp.mean(atom_e, axis=0))
